# baseline (speedup 1.0000x reference)
import functools

import jax
import jax.numpy as jnp
from jax.experimental import pallas as pl
from jax.experimental.pallas import tpu as pltpu


def _conv_body(x1_ref, x2_ref, w_ref, b_ref, o_ref, scr_ref, *, Ho, Wo, C):
    t = pl.program_id(1)

    fc = x2_ref[0, 0]

    @pl.when(t == 0)
    def _init():
        scr_ref[...] = fc

    acc = jnp.zeros((Ho * Wo, C), jnp.float32)
    for kt, f in enumerate((scr_ref[...], x1_ref[0, 0], fc)):
        g_even = f[:, :, :C]
        g_odd = f[:, :, C:]
        zcol = jnp.zeros((g_odd.shape[0], 1, C), f.dtype)
        taps_w = (
            jnp.concatenate([zcol, g_odd[:, :Wo - 1, :]], axis=1),
            g_even,
            g_odd,
        )
        for kw in range(3):
            th = taps_w[kw].reshape(Ho, 2, Wo, C)
            t_even = th[:, 0]
            t_odd = th[:, 1]
            zrow = jnp.zeros((1, Wo, C), f.dtype)
            taps_h = (
                jnp.concatenate([zrow, t_odd[:Ho - 1]], axis=0),
                t_even,
                t_odd,
            )
            for kh in range(3):
                patch = taps_h[kh].reshape(Ho * Wo, C)
                acc = acc + jnp.dot(patch, w_ref[(kt * 3 + kh) * 3 + kw],
                                    preferred_element_type=jnp.float32)
    scr_ref[...] = fc
    acc = acc + b_ref[...]
    o_ref[0, 0] = acc.reshape(Ho, Wo, C).astype(o_ref.dtype)


def kernel(x, weight, bias):
    N, C, T, H, W = x.shape
    K = 3
    To = (T - 1) // 2 + 1
    Ho, Wo = H // 2, W // 2

    Tn = 2 * To - 1
    xl = jnp.transpose(x[:, :, :Tn], (0, 2, 3, 4, 1)).astype(jnp.bfloat16)
    xl = xl.reshape(N, Tn, H, Wo, 2 * C)

    wk = jnp.transpose(weight, (2, 3, 4, 1, 0)).reshape(K * K * K, C, C)
    wk = wk.astype(jnp.bfloat16)
    bk = bias.astype(jnp.float32).reshape(1, C)

    def frame_spec(kt):
        def imap(n, t):
            return (n, jnp.maximum(2 * t + kt - 2, 0), 0, 0, 0)
        return pl.BlockSpec((1, 1, H, Wo, 2 * C), imap)

    out = pl.pallas_call(
        functools.partial(_conv_body, Ho=Ho, Wo=Wo, C=C),
        out_shape=jax.ShapeDtypeStruct((N, To, Ho, Wo, C), x.dtype),
        grid=(N, To),
        in_specs=[frame_spec(1), frame_spec(2),
                  pl.BlockSpec((K * K * K, C, C), lambda n, t: (0, 0, 0)),
                  pl.BlockSpec((1, C), lambda n, t: (0, 0))],
        out_specs=pl.BlockSpec((1, 1, Ho, Wo, C),
                               lambda n, t: (n, t, 0, 0, 0)),
        scratch_shapes=[pltpu.VMEM((H, Wo, 2 * C), jnp.bfloat16)],
        compiler_params=pltpu.CompilerParams(
            dimension_semantics=("parallel", "arbitrary"),
            vmem_limit_bytes=48 * 1024 * 1024),
    )(xl, xl, wk, bk)

    return jnp.transpose(out, (0, 4, 1, 2, 3))


# speedup vs baseline: 1.0086x; 1.0086x over previous
import functools

import jax
import jax.numpy as jnp
from jax.experimental import pallas as pl
from jax.experimental.pallas import tpu as pltpu


def _conv_body(x0_ref, x1_ref, x2_ref, w_ref, b_ref, o_ref, *, Ho, Wo, C):
    acc = jnp.zeros((Ho * Wo, C), jnp.float32)
    for kt, fr in enumerate((x0_ref, x1_ref, x2_ref)):
        f = fr[0, 0]
        g_even = f[:, :, :C]
        g_odd = f[:, :, C:]
        zcol = jnp.zeros((g_odd.shape[0], 1, C), f.dtype)
        taps_w = (
            jnp.concatenate([zcol, g_odd[:, :Wo - 1, :]], axis=1),
            g_even,
            g_odd,
        )
        for kw in range(3):
            th = taps_w[kw].reshape(Ho, 2, Wo, C)
            t_even = th[:, 0]
            t_odd = th[:, 1]
            zrow = jnp.zeros((1, Wo, C), f.dtype)
            taps_h = (
                jnp.concatenate([zrow, t_odd[:Ho - 1]], axis=0),
                t_even,
                t_odd,
            )
            for kh in range(3):
                patch = taps_h[kh].reshape(Ho * Wo, C)
                acc = acc + jnp.dot(patch, w_ref[(kt * 3 + kh) * 3 + kw],
                                    preferred_element_type=jnp.float32)
    acc = acc + b_ref[...]
    o_ref[0, 0] = acc.reshape(Ho, Wo, C).astype(o_ref.dtype)


def kernel(x, weight, bias):
    N, C, T, H, W = x.shape
    K = 3
    To = (T - 1) // 2 + 1
    Ho, Wo = H // 2, W // 2

    Tn = 2 * To - 1
    xl = jnp.transpose(x[:, :, :Tn], (0, 2, 3, 4, 1)).astype(jnp.bfloat16)
    xl = xl.reshape(N, Tn, H, Wo, 2 * C)

    wk = jnp.transpose(weight, (2, 3, 4, 1, 0)).reshape(K * K * K, C, C)
    wk = wk.astype(jnp.bfloat16)
    bk = bias.astype(jnp.float32).reshape(1, C)

    def frame_spec(kt):
        def imap(n, t):
            return (n, jnp.maximum(2 * t + kt - 2, 0), 0, 0, 0)
        return pl.BlockSpec((1, 1, H, Wo, 2 * C), imap)

    out = pl.pallas_call(
        functools.partial(_conv_body, Ho=Ho, Wo=Wo, C=C),
        out_shape=jax.ShapeDtypeStruct((N, To, Ho, Wo, C), x.dtype),
        grid=(N, To),
        in_specs=[frame_spec(0), frame_spec(1), frame_spec(2),
                  pl.BlockSpec((K * K * K, C, C), lambda n, t: (0, 0, 0)),
                  pl.BlockSpec((1, C), lambda n, t: (0, 0))],
        out_specs=pl.BlockSpec((1, 1, Ho, Wo, C),
                               lambda n, t: (n, t, 0, 0, 0)),
        compiler_params=pltpu.CompilerParams(
            dimension_semantics=("parallel", "parallel"),
            vmem_limit_bytes=48 * 1024 * 1024),
    )(xl, xl, xl, wk, bk)

    return jnp.transpose(out, (0, 4, 1, 2, 3))
